# baseline (speedup 1.0000x reference)
import functools

import jax
import jax.numpy as jnp
from jax import lax
from jax.experimental import pallas as pl
from jax.experimental.pallas import tpu as pltpu
from jax.experimental.pallas import tpu_sc as plsc

B = 16384
F = 26
V = 1001
D = 128
ND = 13
H0 = 1024

NC = 2
NS = 16
NW = NC * NS
CH = 128
NSPLIT = 2
BH = B // NSPLIT
BW = BH // NW
NCH = BW // CH
DW = D // 2
NSTEP = F * NCH
NBUF = 5
LOOKAHEAD = 3


def _sc_gather_body(idx_hbm, so_hbm, fo_hbm, emb_out, fo_out,
                    idx_v, rows_v, fo_v, acc_v, sem_g, sem_s):
    wid = lax.axis_index("s") * NC + lax.axis_index("c")
    b0 = wid * BW
    pltpu.sync_copy(fo_hbm, fo_v)
    pltpu.sync_copy(idx_hbm.at[wid], idx_v)
    for j in range(BW // 16):
        acc_v[pl.ds(j * 16, 16)] = jnp.zeros((16,), jnp.float32)

    def store_dst(i):
        f, c = divmod(i, NCH)
        return emb_out.at[pl.ds(b0 + c * CH, CH), pl.ds(f * DW, DW)]

    def fire_gather(i):
        pltpu.async_copy(so_hbm.at[idx_v.at[i]], rows_v.at[i % NBUF], sem_g)

    for i in range(LOOKAHEAD):
        fire_gather(i)
    for i in range(NSTEP):
        if i + LOOKAHEAD < NSTEP:
            if i >= 2:
                pltpu.make_async_copy(
                    rows_v.at[(i - 2) % NBUF], store_dst(i - 2), sem_s).wait()
            fire_gather(i + LOOKAHEAD)
        pltpu.make_async_copy(
            so_hbm.at[idx_v.at[i]], rows_v.at[i % NBUF], sem_g).wait()
        c = i % NCH
        for j in range(CH // 16):
            iv = idx_v[i, pl.ds(j * 16, 16)]
            sl = pl.ds(c * CH + j * 16, 16)
            acc_v[sl] = acc_v[sl] + plsc.load_gather(fo_v, [iv])
        pltpu.async_copy(rows_v.at[i % NBUF], store_dst(i), sem_s)
    for i in range(NSTEP - NBUF, NSTEP):
        pltpu.make_async_copy(
            rows_v.at[i % NBUF], store_dst(i), sem_s).wait()
    pltpu.sync_copy(acc_v, fo_out.at[pl.ds(b0, BW)])


def _sc_gather(idx2d, so_flat, fo_flat):
    mesh = plsc.VectorSubcoreMesh(core_axis_name="c", subcore_axis_name="s")
    return pl.kernel(
        _sc_gather_body,
        out_type=(
            jax.ShapeDtypeStruct((BH, F * DW), jnp.int32),
            jax.ShapeDtypeStruct((BH,), jnp.float32),
        ),
        mesh=mesh,
        compiler_params=pltpu.CompilerParams(needs_layout_passes=False,
                                            use_tc_tiling_on_sc=False),
        scratch_types=[
            pltpu.VMEM((NSTEP, CH), jnp.int32),
            pltpu.VMEM((NBUF, CH, DW), jnp.int32),
            pltpu.VMEM((F * V,), jnp.float32),
            pltpu.VMEM((BW,), jnp.float32),
            pltpu.SemaphoreType.DMA,
            pltpu.SemaphoreType.DMA,
        ],
    )(idx2d, so_flat, fo_flat)


BT = 512


def _tc_body(num_ref, emb_ref, fo_ref, wnum_ref, bnum_ref,
             w0a_ref, w0e_ref, w0o_ref, b0_ref, w1_ref, b1_ref, w2_ref,
             b2_ref, wfm_ref, wh_ref, bout_ref, out_ref):
    num = num_ref[...]
    packed = emb_ref[...]
    lo = jax.lax.bitcast_convert_type(packed << 16, jnp.float32)
    hi = jax.lax.bitcast_convert_type(packed & jnp.int32(-65536), jnp.float32)

    se = jnp.zeros((BT, DW), jnp.float32)
    so = jnp.zeros((BT, DW), jnp.float32)
    q = jnp.zeros((BT, DW), jnp.float32)
    for f in range(F):
        xe = lo[:, f * DW:(f + 1) * DW]
        xo = hi[:, f * DW:(f + 1) * DW]
        se = se + xe
        so = so + xo
        q = q + (xe * xe + xo * xo)
    sumsq = jnp.sum(se * se + so * so, axis=1, keepdims=True)
    sqsum = jnp.sum(q, axis=1, keepdims=True)
    fm2 = 0.5 * (sumsq - sqsum)

    fm1 = jnp.dot(num, wnum_ref[...], preferred_element_type=jnp.float32)
    fm1 = fm1 + bnum_ref[...] + fo_ref[...]
    fm = fm1 + fm2

    bf = jnp.bfloat16
    h = jnp.dot(num, w0a_ref[...], preferred_element_type=jnp.float32)
    h = h + jnp.dot(lo.astype(bf), w0e_ref[...],
                    preferred_element_type=jnp.float32)
    h = h + jnp.dot(hi.astype(bf), w0o_ref[...],
                    preferred_element_type=jnp.float32)
    h = jnp.maximum(h + b0_ref[...], 0.0)
    h = jnp.maximum(jnp.dot(h.astype(bf), w1_ref[...],
                            preferred_element_type=jnp.float32)
                    + b1_ref[...], 0.0)
    h = jnp.maximum(jnp.dot(h.astype(bf), w2_ref[...],
                            preferred_element_type=jnp.float32)
                    + b2_ref[...], 0.0)

    total = fm * wfm_ref[0, 0] + jnp.dot(h, wh_ref[...],
                                         preferred_element_type=jnp.float32)
    total = total + bout_ref[...]
    out_ref[...] = jax.nn.sigmoid(total)


def _tc_mlp(numeric, emb2, fo_sum2, W_num, b_num, W0a, W0e, W0o, b0, W1, b1,
            W2, b2, Wfm, Wh, bout):
    grid = (BH // BT,)

    def full(shape):
        return pl.BlockSpec(shape, lambda *_: tuple(0 for _ in shape))

    return pl.pallas_call(
        _tc_body,
        grid=grid,
        in_specs=[
            pl.BlockSpec((BT, ND), lambda i: (i, 0)),
            pl.BlockSpec((BT, F * DW), lambda i: (i, 0)),
            pl.BlockSpec((BT, 1), lambda i: (i, 0)),
            full((ND, 1)),
            full((1, 1)),
            full((ND, H0)),
            full((F * DW, H0)),
            full((F * DW, H0)),
            full((1, H0)),
            full((H0, 512)),
            full((1, 512)),
            full((512, 256)),
            full((1, 256)),
            full((1, 1)),
            full((256, 1)),
            full((1, 1)),
        ],
        out_specs=pl.BlockSpec((BT, 1), lambda i: (i, 0)),
        out_shape=jax.ShapeDtypeStruct((BH, 1), jnp.float32),
    )(numeric, emb2, fo_sum2, W_num, b_num, W0a, W0e, W0o, b0, W1, b1, W2, b2,
      Wfm, Wh, bout)


@jax.jit
def kernel(numeric, categorical, W_num, b_num, fo_tables, so_tables,
           W0, b0, W1, b1, W2, b2, Wout, bout):
    offs = (jnp.arange(F, dtype=jnp.int32) * V)[:, None]
    so_bf = so_tables.astype(jnp.bfloat16).reshape(F * V, DW, 2)
    so_flat = jax.lax.bitcast_convert_type(so_bf, jnp.int32)
    fo_flat = fo_tables.reshape(F * V)

    W0a = W0[:ND]
    W0b3 = W0[ND:].reshape(F, DW, 2, H0)
    W0e = W0b3[:, :, 0, :].reshape(F * DW, H0).astype(jnp.bfloat16)
    W0o = W0b3[:, :, 1, :].reshape(F * DW, H0).astype(jnp.bfloat16)
    W1 = W1.astype(jnp.bfloat16)
    W2 = W2.astype(jnp.bfloat16)

    gathered = []
    for h in range(NSPLIT):
        cat_h = categorical[h * BH:(h + 1) * BH]
        idx_fw = (cat_h.T + offs).reshape(F, NW, BW)
        idx_w = jnp.transpose(idx_fw, (1, 0, 2)).reshape(NW, NSTEP, CH)
        gathered.append(_sc_gather(idx_w, so_flat, fo_flat))

    outs = []
    for h in range(NSPLIT):
        emb2, fo_sum = gathered[h]
        out = _tc_mlp(numeric[h * BH:(h + 1) * BH], emb2,
                      fo_sum.reshape(BH, 1), W_num, b_num.reshape(1, 1),
                      W0a, W0e, W0o, b0.reshape(1, -1), W1, b1.reshape(1, -1),
                      W2, b2.reshape(1, -1), Wout[0:1], Wout[1:],
                      bout.reshape(1, 1))
        outs.append(out[:, 0])
    return jnp.concatenate(outs)


# speedup vs baseline: 1.2792x; 1.2792x over previous
import functools

import jax
import jax.numpy as jnp
from jax import lax
from jax.experimental import pallas as pl
from jax.experimental.pallas import tpu as pltpu
from jax.experimental.pallas import tpu_sc as plsc

B = 16384
F = 26
V = 1001
D = 128
ND = 13
H0 = 1024

NC = 2
NS = 16
NW = NC * NS
CH = 128
NSPLIT = 2
BH = B // NSPLIT
BW = BH // NW
NCH = BW // CH
DW = D // 2
NSTEP = F * NCH
NBUF = 5
LOOKAHEAD = 3


def _sc_gather_body(idx_hbm, so_hbm, fo_hbm, emb_out, fo_out,
                    idx_v, rows_v, fo_v, acc_v, sem_g, sem_s):
    wid = lax.axis_index("s") * NC + lax.axis_index("c")
    b0 = wid * BW
    pltpu.sync_copy(fo_hbm, fo_v)
    pltpu.sync_copy(idx_hbm.at[pl.ds(wid * NSTEP, NSTEP)], idx_v)
    for j in range(BW // 16):
        acc_v[pl.ds(j * 16, 16)] = jnp.zeros((16,), jnp.float32)

    def store_dst(i):
        f, c = divmod(i, NCH)
        return emb_out.at[f // 2, pl.ds(b0 + c * CH, CH),
                          pl.ds((f % 2) * DW, DW)]

    def fire_gather(i):
        pltpu.async_copy(so_hbm.at[idx_v.at[i]], rows_v.at[i % NBUF], sem_g)

    for i in range(LOOKAHEAD):
        fire_gather(i)
    for i in range(NSTEP):
        if i + LOOKAHEAD < NSTEP:
            if i >= 2:
                pltpu.make_async_copy(
                    rows_v.at[(i - 2) % NBUF], store_dst(i - 2), sem_s).wait()
            fire_gather(i + LOOKAHEAD)
        pltpu.make_async_copy(
            so_hbm.at[idx_v.at[i]], rows_v.at[i % NBUF], sem_g).wait()
        c = i % NCH
        for j in range(CH // 16):
            iv = idx_v[i, pl.ds(j * 16, 16)]
            sl = pl.ds(c * CH + j * 16, 16)
            acc_v[sl] = acc_v[sl] + plsc.load_gather(fo_v, [iv])
        pltpu.async_copy(rows_v.at[i % NBUF], store_dst(i), sem_s)
    for i in range(NSTEP - NBUF, NSTEP):
        pltpu.make_async_copy(
            rows_v.at[i % NBUF], store_dst(i), sem_s).wait()
    pltpu.sync_copy(acc_v, fo_out.at[pl.ds(b0, BW)])


def _sc_gather(idx2d, so_flat, fo_flat):
    mesh = plsc.VectorSubcoreMesh(core_axis_name="c", subcore_axis_name="s")
    return pl.kernel(
        _sc_gather_body,
        out_type=(
            jax.ShapeDtypeStruct((F // 2, BH, 2 * DW), jnp.int32),
            jax.ShapeDtypeStruct((BH,), jnp.float32),
        ),
        mesh=mesh,
        compiler_params=pltpu.CompilerParams(needs_layout_passes=False,
                                            use_tc_tiling_on_sc=False),
        scratch_types=[
            pltpu.VMEM((NSTEP, CH), jnp.int32),
            pltpu.VMEM((NBUF, CH, DW), jnp.int32),
            pltpu.VMEM((F * V,), jnp.float32),
            pltpu.VMEM((BW,), jnp.float32),
            pltpu.SemaphoreType.DMA,
            pltpu.SemaphoreType.DMA,
        ],
    )(idx2d, so_flat, fo_flat)


BT = 512


def _tc_body(num_ref, emb_ref, fo_ref, wnum_ref, bnum_ref,
             w0a_ref, w0e_ref, w0o_ref, b0_ref, w1_ref, b1_ref, w2_ref,
             b2_ref, wfm_ref, wh_ref, bout_ref, out_ref):
    num = num_ref[...]
    x3 = emb_ref[...]
    packed = jnp.concatenate([x3[t] for t in range(F // 2)], axis=1)
    lo = jax.lax.bitcast_convert_type(packed << 16, jnp.float32)
    hi = jax.lax.bitcast_convert_type(packed & jnp.int32(-65536), jnp.float32)

    se = jnp.zeros((BT, DW), jnp.float32)
    so = jnp.zeros((BT, DW), jnp.float32)
    q = jnp.zeros((BT, DW), jnp.float32)
    for f in range(F):
        base = (f // 2) * 128 + (f % 2) * DW
        xe = lo[:, base:base + DW]
        xo = hi[:, base:base + DW]
        se = se + xe
        so = so + xo
        q = q + (xe * xe + xo * xo)
    sumsq = jnp.sum(se * se + so * so, axis=1, keepdims=True)
    sqsum = jnp.sum(q, axis=1, keepdims=True)
    fm2 = 0.5 * (sumsq - sqsum)

    fm1 = jnp.dot(num, wnum_ref[...], preferred_element_type=jnp.float32)
    fm1 = fm1 + bnum_ref[...] + fo_ref[...]
    fm = fm1 + fm2

    bf = jnp.bfloat16
    h = jnp.dot(num, w0a_ref[...], preferred_element_type=jnp.float32)
    h = h + jnp.dot(lo.astype(bf), w0e_ref[...],
                    preferred_element_type=jnp.float32)
    h = h + jnp.dot(hi.astype(bf), w0o_ref[...],
                    preferred_element_type=jnp.float32)
    h = jnp.maximum(h + b0_ref[...], 0.0)
    h = jnp.maximum(jnp.dot(h.astype(bf), w1_ref[...],
                            preferred_element_type=jnp.float32)
                    + b1_ref[...], 0.0)
    h = jnp.maximum(jnp.dot(h.astype(bf), w2_ref[...],
                            preferred_element_type=jnp.float32)
                    + b2_ref[...], 0.0)

    total = fm * wfm_ref[0, 0] + jnp.dot(h, wh_ref[...],
                                         preferred_element_type=jnp.float32)
    total = total + bout_ref[...]
    out_ref[...] = jax.nn.sigmoid(total)


def _tc_mlp(numeric, emb2, fo_sum2, W_num, b_num, W0a, W0e, W0o, b0, W1, b1,
            W2, b2, Wfm, Wh, bout):
    grid = (BH // BT,)

    def full(shape):
        return pl.BlockSpec(shape, lambda *_: tuple(0 for _ in shape))

    return pl.pallas_call(
        _tc_body,
        grid=grid,
        in_specs=[
            pl.BlockSpec((BT, ND), lambda i: (i, 0)),
            pl.BlockSpec((F // 2, BT, 2 * DW), lambda i: (0, i, 0)),
            pl.BlockSpec((BT, 1), lambda i: (i, 0)),
            full((ND, 1)),
            full((1, 1)),
            full((ND, H0)),
            full((F * DW, H0)),
            full((F * DW, H0)),
            full((1, H0)),
            full((H0, 512)),
            full((1, 512)),
            full((512, 256)),
            full((1, 256)),
            full((1, 1)),
            full((256, 1)),
            full((1, 1)),
        ],
        out_specs=pl.BlockSpec((BT, 1), lambda i: (i, 0)),
        out_shape=jax.ShapeDtypeStruct((BH, 1), jnp.float32),
    )(numeric, emb2, fo_sum2, W_num, b_num, W0a, W0e, W0o, b0, W1, b1, W2, b2,
      Wfm, Wh, bout)


@jax.jit
def kernel(numeric, categorical, W_num, b_num, fo_tables, so_tables,
           W0, b0, W1, b1, W2, b2, Wout, bout):
    offs = (jnp.arange(F, dtype=jnp.int32) * V)[:, None]
    so_bf = so_tables.astype(jnp.bfloat16).reshape(F * V, DW, 2)
    so_flat = jax.lax.bitcast_convert_type(so_bf, jnp.int32)
    fo_flat = fo_tables.reshape(F * V)

    W0a = W0[:ND]
    W0b4 = W0[ND:].reshape(F // 2, 2, DW, 2, H0)
    W0e = jnp.transpose(W0b4[:, :, :, 0, :], (0, 1, 2, 3)).reshape(
        F * DW, H0).astype(jnp.bfloat16)
    W0o = jnp.transpose(W0b4[:, :, :, 1, :], (0, 1, 2, 3)).reshape(
        F * DW, H0).astype(jnp.bfloat16)
    W1 = W1.astype(jnp.bfloat16)
    W2 = W2.astype(jnp.bfloat16)

    gathered = []
    for h in range(NSPLIT):
        cat_h = categorical[h * BH:(h + 1) * BH]
        idx_fw = (cat_h.T + offs).reshape(F, NW, BW)
        idx_w = jnp.transpose(idx_fw, (1, 0, 2)).reshape(NW * NSTEP, CH)
        gathered.append(_sc_gather(idx_w, so_flat, fo_flat))

    outs = []
    for h in range(NSPLIT):
        emb2, fo_sum = gathered[h]
        out = _tc_mlp(numeric[h * BH:(h + 1) * BH], emb2,
                      fo_sum.reshape(BH, 1), W_num, b_num.reshape(1, 1),
                      W0a, W0e, W0o, b0.reshape(1, -1), W1, b1.reshape(1, -1),
                      W2, b2.reshape(1, -1), Wout[0:1], Wout[1:],
                      bout.reshape(1, 1))
        outs.append(out[:, 0])
    return jnp.concatenate(outs)


# speedup vs baseline: 1.6789x; 1.3124x over previous
import functools

import jax
import jax.numpy as jnp
from jax import lax
from jax.experimental import pallas as pl
from jax.experimental.pallas import tpu as pltpu
from jax.experimental.pallas import tpu_sc as plsc

B = 16384
F = 26
V = 1001
D = 128
ND = 13
H0 = 1024
FV = F * V

NC = 2
NS = 16
NW = NC * NS
CH = 128
NSPLIT = 2
BH = B // NSPLIT
BW = BH // NW
NCH = BW // CH
NSTEP = F * NCH
NBUF = 5
LOOKAHEAD = 3
DW = D // 2

PROWS = 64
NFULL = 12
NTAIL = FV - NW * NFULL * PROWS - 22 * PROWS


def _sc_pack_body(so_hbm, out_hbm, in_v, out_v, sem_i, sem_o):
    wid = lax.axis_index("s") * NC + lax.axis_index("c")

    def fire_in(k):
        row0 = (wid + k * NW) * PROWS
        pltpu.async_copy(so_hbm.at[pl.ds(row0, PROWS)], in_v.at[k % 2], sem_i)

    def pack_rows(src, dst, nrows):
        def row_body(r, _):
            for g in range(4):
                a = src[r, pl.ds(g * 32, 16)]
                b = src[r, pl.ds(g * 32 + 16, 16)]
                p = plsc.pack(a, b, format=plsc.PackFormat.INTERLEAVED)
                dst[r, pl.ds(g * 16, 16)] = plsc.bitcast(p, jnp.int32)
            return 0
        lax.fori_loop(0, nrows, row_body, 0)

    fire_in(0)
    for k in range(NFULL):
        if k + 1 < NFULL:
            fire_in(k + 1)
        pltpu.make_async_copy(
            so_hbm.at[pl.ds(0, PROWS)], in_v.at[k % 2], sem_i).wait()
        pack_rows(in_v.at[k % 2], out_v.at[k % 2], PROWS)
        row0 = (wid + k * NW) * PROWS
        pltpu.async_copy(out_v.at[k % 2],
                         out_hbm.at[pl.ds(row0, PROWS)], sem_o)
        if k >= 1:
            pltpu.make_async_copy(
                out_v.at[k % 2], out_hbm.at[pl.ds(0, PROWS)], sem_o).wait()
    pltpu.make_async_copy(
        out_v.at[0], out_hbm.at[pl.ds(0, PROWS)], sem_o).wait()

    t0 = NW * NFULL * PROWS

    @pl.when(wid < 22)
    def _():
        row0 = t0 + wid * PROWS
        pltpu.sync_copy(so_hbm.at[pl.ds(row0, PROWS)], in_v.at[0])
        pack_rows(in_v.at[0], out_v.at[0], PROWS)
        pltpu.sync_copy(out_v.at[0], out_hbm.at[pl.ds(row0, PROWS)])

    @pl.when(wid == 22)
    def _():
        row0 = t0 + 22 * PROWS
        pltpu.sync_copy(so_hbm.at[pl.ds(row0, NTAIL)],
                        in_v.at[0, pl.ds(0, NTAIL)])
        pack_rows(in_v.at[0], out_v.at[0], NTAIL)
        pltpu.sync_copy(out_v.at[0, pl.ds(0, NTAIL)],
                        out_hbm.at[pl.ds(row0, NTAIL)])


def _sc_pack(so_flat):
    mesh = plsc.VectorSubcoreMesh(core_axis_name="c", subcore_axis_name="s")
    return pl.kernel(
        _sc_pack_body,
        out_type=jax.ShapeDtypeStruct((FV, DW), jnp.int32),
        mesh=mesh,
        compiler_params=pltpu.CompilerParams(needs_layout_passes=False,
                                             use_tc_tiling_on_sc=False),
        scratch_types=[
            pltpu.VMEM((2, PROWS, D), jnp.float32),
            pltpu.VMEM((2, PROWS, DW), jnp.int32),
            pltpu.SemaphoreType.DMA,
            pltpu.SemaphoreType.DMA,
        ],
    )(so_flat)


def _sc_gather_body(h, cat_hbm, so_hbm, fo_hbm, emb_out, fo_out,
                    idx_v, rows_v, fo_v, acc_v, sem_g, sem_s, sem_i):
    wid = lax.axis_index("s") * NC + lax.axis_index("c")
    b0 = wid * BW
    pltpu.sync_copy(fo_hbm, fo_v)
    kb = (h * BH + b0) // CH
    for f in range(F):
        pltpu.async_copy(cat_hbm.at[pl.ds(f * (B // CH) + kb, NCH)],
                         idx_v.at[pl.ds(f * NCH, NCH)], sem_i)
    for j in range(BW // 16):
        acc_v[pl.ds(j * 16, 16)] = jnp.zeros((16,), jnp.float32)
    for f in range(F):
        pltpu.make_async_copy(
            cat_hbm.at[pl.ds(0, NCH)], idx_v.at[pl.ds(0, NCH)], sem_i).wait()
    for i in range(NSTEP):
        f = i // NCH
        for j in range(CH // 16):
            sl = pl.ds(j * 16, 16)
            idx_v[i, sl] = idx_v[i, sl] + jnp.int32(f * V)

    def store_dst(i):
        f, c = divmod(i, NCH)
        return emb_out.at[f // 2, pl.ds(b0 + c * CH, CH),
                          pl.ds((f % 2) * DW, DW)]

    def fire_gather(i):
        pltpu.async_copy(so_hbm.at[idx_v.at[i]], rows_v.at[i % NBUF], sem_g)

    for i in range(LOOKAHEAD):
        fire_gather(i)
    for i in range(NSTEP):
        if i + LOOKAHEAD < NSTEP:
            if i >= 2:
                pltpu.make_async_copy(
                    rows_v.at[(i - 2) % NBUF], store_dst(i - 2), sem_s).wait()
            fire_gather(i + LOOKAHEAD)
        pltpu.make_async_copy(
            so_hbm.at[idx_v.at[i]], rows_v.at[i % NBUF], sem_g).wait()
        c = i % NCH
        for j in range(CH // 16):
            iv = idx_v[i, pl.ds(j * 16, 16)]
            sl = pl.ds(c * CH + j * 16, 16)
            acc_v[sl] = acc_v[sl] + plsc.load_gather(fo_v, [iv])
        pltpu.async_copy(rows_v.at[i % NBUF], store_dst(i), sem_s)
    for i in range(NSTEP - NBUF, NSTEP):
        pltpu.make_async_copy(
            rows_v.at[i % NBUF], store_dst(i), sem_s).wait()
    pltpu.sync_copy(acc_v, fo_out.at[pl.ds(b0, BW)])


def _sc_gather(cat_r, so_packed, fo_flat, h):
    mesh = plsc.VectorSubcoreMesh(core_axis_name="c", subcore_axis_name="s")
    return pl.kernel(
        functools.partial(_sc_gather_body, h),
        out_type=(
            jax.ShapeDtypeStruct((F // 2, BH, 2 * DW), jnp.int32),
            jax.ShapeDtypeStruct((BH,), jnp.float32),
        ),
        mesh=mesh,
        compiler_params=pltpu.CompilerParams(needs_layout_passes=False,
                                             use_tc_tiling_on_sc=False),
        scratch_types=[
            pltpu.VMEM((NSTEP, CH), jnp.int32),
            pltpu.VMEM((NBUF, CH, DW), jnp.int32),
            pltpu.VMEM((FV,), jnp.float32),
            pltpu.VMEM((BW,), jnp.float32),
            pltpu.SemaphoreType.DMA,
            pltpu.SemaphoreType.DMA,
            pltpu.SemaphoreType.DMA,
        ],
    )(cat_r, so_packed, fo_flat)


BT = 512


def _tc_body(num_ref, emb_ref, fo_ref, wnum_ref, bnum_ref,
             w0a_ref, w0e_ref, w0o_ref, b0_ref, w1_ref, b1_ref, w2_ref,
             b2_ref, wfm_ref, wh_ref, bout_ref, out_ref):
    num = num_ref[...]
    x3 = emb_ref[...]
    packed = jnp.concatenate([x3[t] for t in range(F // 2)], axis=1)
    lo = jax.lax.bitcast_convert_type(packed << 16, jnp.float32)
    hi = jax.lax.bitcast_convert_type(packed & jnp.int32(-65536), jnp.float32)

    se = jnp.zeros((BT, DW), jnp.float32)
    so = jnp.zeros((BT, DW), jnp.float32)
    q = jnp.zeros((BT, DW), jnp.float32)
    for f in range(F):
        base = (f // 2) * 128 + (f % 2) * DW
        xe = lo[:, base:base + DW]
        xo = hi[:, base:base + DW]
        se = se + xe
        so = so + xo
        q = q + (xe * xe + xo * xo)
    sumsq = jnp.sum(se * se + so * so, axis=1, keepdims=True)
    sqsum = jnp.sum(q, axis=1, keepdims=True)
    fm2 = 0.5 * (sumsq - sqsum)

    fm1 = jnp.dot(num, wnum_ref[...], preferred_element_type=jnp.float32)
    fm1 = fm1 + bnum_ref[...] + fo_ref[...]
    fm = fm1 + fm2

    bf = jnp.bfloat16
    h = jnp.dot(num, w0a_ref[...], preferred_element_type=jnp.float32)
    h = h + jnp.dot(lo.astype(bf), w0e_ref[...],
                    preferred_element_type=jnp.float32)
    h = h + jnp.dot(hi.astype(bf), w0o_ref[...],
                    preferred_element_type=jnp.float32)
    h = jnp.maximum(h + b0_ref[...], 0.0)
    h = jnp.maximum(jnp.dot(h.astype(bf), w1_ref[...],
                            preferred_element_type=jnp.float32)
                    + b1_ref[...], 0.0)
    h = jnp.maximum(jnp.dot(h.astype(bf), w2_ref[...],
                            preferred_element_type=jnp.float32)
                    + b2_ref[...], 0.0)

    total = fm * wfm_ref[0, 0] + jnp.dot(h, wh_ref[...],
                                         preferred_element_type=jnp.float32)
    total = total + bout_ref[...]
    out_ref[...] = jax.nn.sigmoid(total)


def _tc_mlp(numeric, emb2, fo_sum2, W_num, b_num, W0a, W0e, W0o, b0, W1, b1,
            W2, b2, Wfm, Wh, bout):
    grid = (BH // BT,)

    def full(shape):
        return pl.BlockSpec(shape, lambda *_: tuple(0 for _ in shape))

    return pl.pallas_call(
        _tc_body,
        grid=grid,
        in_specs=[
            pl.BlockSpec((BT, ND), lambda i: (i, 0)),
            pl.BlockSpec((F // 2, BT, 2 * DW), lambda i: (0, i, 0)),
            pl.BlockSpec((BT, 1), lambda i: (i, 0)),
            full((ND, 1)),
            full((1, 1)),
            full((ND, H0)),
            full((F * DW, H0)),
            full((F * DW, H0)),
            full((1, H0)),
            full((H0, 512)),
            full((1, 512)),
            full((512, 256)),
            full((1, 256)),
            full((1, 1)),
            full((256, 1)),
            full((1, 1)),
        ],
        out_specs=pl.BlockSpec((BT, 1), lambda i: (i, 0)),
        out_shape=jax.ShapeDtypeStruct((BH, 1), jnp.float32),
    )(numeric, emb2, fo_sum2, W_num, b_num, W0a, W0e, W0o, b0, W1, b1, W2, b2,
      Wfm, Wh, bout)


@jax.jit
def kernel(numeric, categorical, W_num, b_num, fo_tables, so_tables,
           W0, b0, W1, b1, W2, b2, Wout, bout):
    so_flat = so_tables.reshape(FV, D)
    fo_flat = fo_tables.reshape(FV)
    cat_r = categorical.T.reshape(F * (B // CH), CH)

    so_packed = _sc_pack(so_flat)

    W0a = W0[:ND]
    W0r = W0[ND:].reshape(F // 2, 2, 4, 2, 16, H0)
    W0e = W0r[:, :, :, 0, :, :].reshape(F * DW, H0).astype(jnp.bfloat16)
    W0o = W0r[:, :, :, 1, :, :].reshape(F * DW, H0).astype(jnp.bfloat16)
    W1 = W1.astype(jnp.bfloat16)
    W2 = W2.astype(jnp.bfloat16)

    gathered = []
    for h in range(NSPLIT):
        gathered.append(_sc_gather(cat_r, so_packed, fo_flat, h))

    outs = []
    for h in range(NSPLIT):
        emb2, fo_sum = gathered[h]
        out = _tc_mlp(numeric[h * BH:(h + 1) * BH], emb2,
                      fo_sum.reshape(BH, 1), W_num, b_num.reshape(1, 1),
                      W0a, W0e, W0o, b0.reshape(1, -1), W1, b1.reshape(1, -1),
                      W2, b2.reshape(1, -1), Wout[0:1], Wout[1:],
                      bout.reshape(1, 1))
        outs.append(out[:, 0])
    return jnp.concatenate(outs)


# speedup vs baseline: 1.6894x; 1.0063x over previous
import functools

import jax
import jax.numpy as jnp
from jax import lax
from jax.experimental import pallas as pl
from jax.experimental.pallas import tpu as pltpu
from jax.experimental.pallas import tpu_sc as plsc

B = 16384
F = 26
V = 1001
D = 128
ND = 13
H0 = 1024
FV = F * V

NC = 2
NS = 16
NW = NC * NS
CH = 128
NSPLIT = 2
BH = B // NSPLIT
BW = BH // NW
NCH = BW // CH
NSTEP = F * NCH
NBUF = 5
LOOKAHEAD = 3
DW = D // 2

PROWS = 64
NFULL = 12
NTAIL = FV - NW * NFULL * PROWS - 22 * PROWS


def _sc_pack_body(so_hbm, out_hbm, in_v, out_v, sem_i, sem_o):
    wid = lax.axis_index("s") * NC + lax.axis_index("c")

    def fire_in(k):
        row0 = (wid + k * NW) * PROWS
        pltpu.async_copy(so_hbm.at[pl.ds(row0, PROWS)], in_v.at[k % 2], sem_i)

    def pack_rows(src, dst, nrows):
        def row_body(r, _):
            for g in range(4):
                a = src[r, pl.ds(g * 32, 16)]
                b = src[r, pl.ds(g * 32 + 16, 16)]
                p = plsc.pack(a, b, format=plsc.PackFormat.INTERLEAVED)
                dst[r, pl.ds(g * 16, 16)] = plsc.bitcast(p, jnp.int32)
            return 0
        lax.fori_loop(0, nrows, row_body, 0)

    fire_in(0)
    for k in range(NFULL):
        if k + 1 < NFULL:
            fire_in(k + 1)
        pltpu.make_async_copy(
            so_hbm.at[pl.ds(0, PROWS)], in_v.at[k % 2], sem_i).wait()
        pack_rows(in_v.at[k % 2], out_v.at[k % 2], PROWS)
        row0 = (wid + k * NW) * PROWS
        pltpu.async_copy(out_v.at[k % 2],
                         out_hbm.at[pl.ds(row0, PROWS)], sem_o)
        if k >= 1:
            pltpu.make_async_copy(
                out_v.at[k % 2], out_hbm.at[pl.ds(0, PROWS)], sem_o).wait()
    pltpu.make_async_copy(
        out_v.at[0], out_hbm.at[pl.ds(0, PROWS)], sem_o).wait()

    t0 = NW * NFULL * PROWS

    @pl.when(wid < 22)
    def _():
        row0 = t0 + wid * PROWS
        pltpu.sync_copy(so_hbm.at[pl.ds(row0, PROWS)], in_v.at[0])
        pack_rows(in_v.at[0], out_v.at[0], PROWS)
        pltpu.sync_copy(out_v.at[0], out_hbm.at[pl.ds(row0, PROWS)])

    @pl.when(wid == 22)
    def _():
        row0 = t0 + 22 * PROWS
        pltpu.sync_copy(so_hbm.at[pl.ds(row0, NTAIL)],
                        in_v.at[0, pl.ds(0, NTAIL)])
        pack_rows(in_v.at[0], out_v.at[0], NTAIL)
        pltpu.sync_copy(out_v.at[0, pl.ds(0, NTAIL)],
                        out_hbm.at[pl.ds(row0, NTAIL)])


def _sc_pack(so_flat):
    mesh = plsc.VectorSubcoreMesh(core_axis_name="c", subcore_axis_name="s")
    return pl.kernel(
        _sc_pack_body,
        out_type=jax.ShapeDtypeStruct((FV, DW), jnp.int32),
        mesh=mesh,
        compiler_params=pltpu.CompilerParams(needs_layout_passes=False,
                                             use_tc_tiling_on_sc=False),
        scratch_types=[
            pltpu.VMEM((2, PROWS, D), jnp.float32),
            pltpu.VMEM((2, PROWS, DW), jnp.int32),
            pltpu.SemaphoreType.DMA,
            pltpu.SemaphoreType.DMA,
        ],
    )(so_flat)


def _sc_gather_body(h, cat_hbm, so_hbm, fo_hbm, emb_out, fo_out,
                    idx_v, rows_v, fo_v, acc_v, sem_g, sem_s, sem_i):
    wid = lax.axis_index("s") * NC + lax.axis_index("c")
    b0 = wid * BW
    pltpu.sync_copy(fo_hbm, fo_v)
    kb = (h * BH + b0) // CH
    for f in range(F):
        pltpu.async_copy(cat_hbm.at[pl.ds(f * (B // CH) + kb, NCH)],
                         idx_v.at[pl.ds(f * NCH, NCH)], sem_i)
    for j in range(BW // 16):
        acc_v[pl.ds(j * 16, 16)] = jnp.zeros((16,), jnp.float32)
    for f in range(F):
        pltpu.make_async_copy(
            cat_hbm.at[pl.ds(0, NCH)], idx_v.at[pl.ds(0, NCH)], sem_i).wait()
    for i in range(NSTEP):
        f = i // NCH
        for j in range(CH // 16):
            sl = pl.ds(j * 16, 16)
            idx_v[i, sl] = idx_v[i, sl] + jnp.int32(f * V)

    def store_dst(i):
        f, c = divmod(i, NCH)
        return emb_out.at[f // 2, pl.ds(b0 + c * CH, CH),
                          pl.ds((f % 2) * DW, DW)]

    def fire_gather(i):
        pltpu.async_copy(so_hbm.at[idx_v.at[i]], rows_v.at[i % NBUF], sem_g)

    for i in range(LOOKAHEAD):
        fire_gather(i)
    for i in range(NSTEP):
        if i + LOOKAHEAD < NSTEP:
            if i >= 2:
                pltpu.make_async_copy(
                    rows_v.at[(i - 2) % NBUF], store_dst(i - 2), sem_s).wait()
            fire_gather(i + LOOKAHEAD)
        pltpu.make_async_copy(
            so_hbm.at[idx_v.at[i]], rows_v.at[i % NBUF], sem_g).wait()
        c = i % NCH
        for j in range(CH // 16):
            iv = idx_v[i, pl.ds(j * 16, 16)]
            sl = pl.ds(c * CH + j * 16, 16)
            acc_v[sl] = acc_v[sl] + plsc.load_gather(fo_v, [iv])
        pltpu.async_copy(rows_v.at[i % NBUF], store_dst(i), sem_s)
    for i in range(NSTEP - NBUF, NSTEP):
        pltpu.make_async_copy(
            rows_v.at[i % NBUF], store_dst(i), sem_s).wait()
    pltpu.sync_copy(acc_v, fo_out.at[pl.ds(b0, BW)])


def _sc_gather(cat_r, so_packed, fo_flat, h):
    mesh = plsc.VectorSubcoreMesh(core_axis_name="c", subcore_axis_name="s")
    return pl.kernel(
        functools.partial(_sc_gather_body, h),
        out_type=(
            jax.ShapeDtypeStruct((F // 2, BH, 2 * DW), jnp.int32),
            jax.ShapeDtypeStruct((BH,), jnp.float32),
        ),
        mesh=mesh,
        compiler_params=pltpu.CompilerParams(needs_layout_passes=False,
                                             use_tc_tiling_on_sc=False),
        scratch_types=[
            pltpu.VMEM((NSTEP, CH), jnp.int32),
            pltpu.VMEM((NBUF, CH, DW), jnp.int32),
            pltpu.VMEM((FV,), jnp.float32),
            pltpu.VMEM((BW,), jnp.float32),
            pltpu.SemaphoreType.DMA,
            pltpu.SemaphoreType.DMA,
            pltpu.SemaphoreType.DMA,
        ],
    )(cat_r, so_packed, fo_flat)


BT = 512


def _tc_body(num_ref, emb_ref, fo_ref, wnum_ref, bnum_ref,
             w0a_ref, waug_ref, b0_ref, w1_ref, b1_ref, w2_ref,
             b2_ref, wfm_ref, wh_ref, bout_ref, out_ref):
    num = num_ref[...]
    x3 = emb_ref[...]
    packed = jnp.concatenate([x3[t] for t in range(F // 2)], axis=1)
    lo = jax.lax.bitcast_convert_type(packed << 16, jnp.float32)
    hi = jax.lax.bitcast_convert_type(packed & jnp.int32(-65536), jnp.float32)

    q2 = lo * lo + hi * hi
    sqsum = jnp.sum(q2, axis=1, keepdims=True)

    se = jnp.zeros((BT, DW), jnp.float32)
    so = jnp.zeros((BT, DW), jnp.float32)
    for f in range(F):
        base = (f // 2) * 128 + (f % 2) * DW
        se = se + lo[:, base:base + DW]
        so = so + hi[:, base:base + DW]
    sumsq = jnp.sum(se * se + so * so, axis=1, keepdims=True)
    fm2 = 0.5 * (sumsq - sqsum)

    bf = jnp.bfloat16
    X = jnp.concatenate([lo.astype(bf), hi.astype(bf)], axis=1)
    h = jnp.dot(X, waug_ref[...], preferred_element_type=jnp.float32)

    fm1 = jnp.dot(num, wnum_ref[...], preferred_element_type=jnp.float32)
    fm1 = fm1 + bnum_ref[...] + fo_ref[...]
    fm = fm1 + fm2

    h = h + jnp.dot(num, w0a_ref[...], preferred_element_type=jnp.float32)
    h = jnp.maximum(h + b0_ref[...], 0.0)
    h = jnp.maximum(jnp.dot(h.astype(bf), w1_ref[...],
                            preferred_element_type=jnp.float32)
                    + b1_ref[...], 0.0)
    h = jnp.maximum(jnp.dot(h.astype(bf), w2_ref[...],
                            preferred_element_type=jnp.float32)
                    + b2_ref[...], 0.0)

    total = fm * wfm_ref[0, 0] + jnp.dot(h, wh_ref[...],
                                         preferred_element_type=jnp.float32)
    total = total + bout_ref[...]
    out_ref[...] = jax.nn.sigmoid(total)


def _tc_mlp(numeric, emb2, fo_sum2, W_num, b_num, W0a, Waug, b0, W1, b1,
            W2, b2, Wfm, Wh, bout):
    grid = (BH // BT,)

    def full(shape):
        return pl.BlockSpec(shape, lambda *_: tuple(0 for _ in shape))

    return pl.pallas_call(
        _tc_body,
        grid=grid,
        in_specs=[
            pl.BlockSpec((BT, ND), lambda i: (i, 0)),
            pl.BlockSpec((F // 2, BT, 2 * DW), lambda i: (0, i, 0)),
            pl.BlockSpec((BT, 1), lambda i: (i, 0)),
            full((ND, 1)),
            full((1, 1)),
            full((ND, H0)),
            full((2 * F * DW, H0)),
            full((1, H0)),
            full((H0, 512)),
            full((1, 512)),
            full((512, 256)),
            full((1, 256)),
            full((1, 1)),
            full((256, 1)),
            full((1, 1)),
        ],
        out_specs=pl.BlockSpec((BT, 1), lambda i: (i, 0)),
        out_shape=jax.ShapeDtypeStruct((BH, 1), jnp.float32),
    )(numeric, emb2, fo_sum2, W_num, b_num, W0a, Waug, b0, W1, b1, W2, b2,
      Wfm, Wh, bout)


@jax.jit
def kernel(numeric, categorical, W_num, b_num, fo_tables, so_tables,
           W0, b0, W1, b1, W2, b2, Wout, bout):
    so_flat = so_tables.reshape(FV, D)
    fo_flat = fo_tables.reshape(FV)
    cat_r = categorical.T.reshape(F * (B // CH), CH)

    so_packed = _sc_pack(so_flat)

    W0a = W0[:ND]
    W0r = W0[ND:].reshape(F // 2, 2, 4, 2, 16, H0)
    W0e = W0r[:, :, :, 0, :, :].reshape(F * DW, H0)
    W0o = W0r[:, :, :, 1, :, :].reshape(F * DW, H0)
    W1 = W1.astype(jnp.bfloat16)
    W2 = W2.astype(jnp.bfloat16)
    Waug = jnp.concatenate([W0e, W0o], axis=0).astype(jnp.bfloat16)

    gathered = []
    for h in range(NSPLIT):
        gathered.append(_sc_gather(cat_r, so_packed, fo_flat, h))

    outs = []
    for h in range(NSPLIT):
        emb2, fo_sum = gathered[h]
        out = _tc_mlp(numeric[h * BH:(h + 1) * BH], emb2,
                      fo_sum.reshape(BH, 1), W_num, b_num.reshape(1, 1),
                      W0a, Waug, b0.reshape(1, -1), W1, b1.reshape(1, -1),
                      W2, b2.reshape(1, -1), Wout[0:1], Wout[1:],
                      bout.reshape(1, 1))
        outs.append(out[:, 0])
    return jnp.concatenate(outs)
